# Initial kernel scaffold; baseline (speedup 1.0000x reference)
import jax
import jax.numpy as jnp
from jax.experimental import pallas as pl


def kernel(input_ids, vectors, word_table, pos_table):
    raise NotImplementedError("write your pallas kernel here")


# speedup vs baseline: 2.4197x; 2.4197x over previous
import functools

import jax
import jax.numpy as jnp
from jax import lax
from jax.experimental import pallas as pl
from jax.experimental.pallas import tpu as pltpu
from jax.experimental.pallas import tpu_sc as plsc

VOCAB = 1000000
HID = 64
MAXPOS = 200
B = 4096
L = 200

NC = 2
NS = 16
NW = NC * NS
ROWS_PER_W = B // NW
C = 4
NCHUNK = ROWS_PER_W // C
SUB = 80
NSUB = C * L // SUB


def _body(ids_hbm, vec_hbm, wtab_hbm, ptab_hbm, out_hbm,
          idx_v, rows_v, pos_v, vecbuf_v, gsem):
  wid = lax.axis_index("s") * NC + lax.axis_index("c")
  base = wid * ROWS_PER_W

  pltpu.sync_copy(vec_hbm.at[pl.ds(base, ROWS_PER_W)], vecbuf_v)
  pltpu.sync_copy(vecbuf_v, out_hbm.at[pl.ds(base, ROWS_PER_W), 0])

  pltpu.sync_copy(ptab_hbm.at[pl.ds(1, L)], pos_v)

  @pl.loop(0, NCHUNK)
  def _chunk(g):
    row0 = base + g * C
    pltpu.sync_copy(ids_hbm.at[pl.ds(row0 * L, C * L)], idx_v)
    for k in range(NSUB):
      pltpu.async_copy(
          wtab_hbm.at[idx_v.at[pl.ds(k * SUB, SUB)]],
          rows_v.at[pl.ds(k * SUB, SUB)],
          gsem,
      ).wait()

    @pl.loop(0, L)
    def _pos(j):
      for q in range(HID // 16):
        p = pos_v[j, pl.ds(q * 16, 16)]
        for c in range(C):
          r = c * L + j
          rows_v[r, pl.ds(q * 16, 16)] += p

    for c in range(C):
      pltpu.sync_copy(rows_v.at[pl.ds(c * L, L)],
                      out_hbm.at[row0 + c, pl.ds(1, L)])


def kernel(input_ids, vectors, word_table, pos_table):
  ids_flat = input_ids.reshape(B * L)
  mesh = plsc.VectorSubcoreMesh(core_axis_name="c", subcore_axis_name="s",
                                num_cores=NC, num_subcores=NS)
  out = pl.kernel(
      _body,
      out_type=jax.ShapeDtypeStruct((B, MAXPOS + 1, HID), jnp.float32),
      mesh=mesh,
      compiler_params=pltpu.CompilerParams(use_tc_tiling_on_sc=False),
      scratch_types=[
          pltpu.VMEM((C * L,), jnp.int32),
          pltpu.VMEM((C * L, HID), jnp.float32),
          pltpu.VMEM((L, HID), jnp.float32),
          pltpu.VMEM((ROWS_PER_W, HID), jnp.float32),
          pltpu.SemaphoreType.DMA,
      ],
  )(ids_flat, vectors, word_table, pos_table)
  return out


# speedup vs baseline: 2.9323x; 1.2118x over previous
import functools

import jax
import jax.numpy as jnp
from jax import lax
from jax.experimental import pallas as pl
from jax.experimental.pallas import tpu as pltpu
from jax.experimental.pallas import tpu_sc as plsc

VOCAB = 1000000
HID = 64
MAXPOS = 200
B = 4096
L = 200

NC = 2
NS = 16
NW = NC * NS
ROWS_PER_W = B // NW
C = 2
NCHUNK = ROWS_PER_W // C
SUB = 80
NSUB = C * L // SUB


def _body(ids_hbm, vec_hbm, wtab_hbm, ptab_hbm, out_hbm,
          idx0, idx1, rows0, rows1, pos_v, vecbuf_v,
          isem0, isem1, gsem0, gsem1, osem0, osem1, vsem):
  wid = lax.axis_index("s") * NC + lax.axis_index("c")
  base = wid * ROWS_PER_W

  idx = (idx0, idx1)
  rows = (rows0, rows1)
  isem = (isem0, isem1)
  gsem = (gsem0, gsem1)
  osem = (osem0, osem1)

  def issue_idx(g, b):
    pltpu.async_copy(ids_hbm.at[pl.ds((base + g * C) * L, C * L)],
                     idx[b], isem[b])

  def wait_idx(b):
    pltpu.make_async_copy(ids_hbm.at[pl.ds(0, C * L)], idx[b], isem[b]).wait()

  def issue_gathers(b):
    for k in range(NSUB):
      pltpu.async_copy(
          wtab_hbm.at[idx[b].at[pl.ds(k * SUB, SUB)]],
          rows[b].at[pl.ds(k * SUB, SUB)],
          gsem[b],
      )

  def wait_gathers(b):
    pltpu.make_async_copy(wtab_hbm.at[pl.ds(0, C * L)], rows[b],
                          gsem[b]).wait()

  def issue_out(g, b):
    for c in range(C):
      pltpu.async_copy(rows[b].at[pl.ds(c * L, L)],
                       out_hbm.at[base + g * C + c, pl.ds(1, L)], osem[b])

  def wait_out(b):
    for c in range(C):
      pltpu.make_async_copy(rows[b].at[pl.ds(c * L, L)],
                            out_hbm.at[base + c, pl.ds(1, L)], osem[b]).wait()

  vdesc = pltpu.async_copy(vec_hbm.at[pl.ds(base, ROWS_PER_W)], vecbuf_v, vsem)

  pltpu.sync_copy(ptab_hbm.at[pl.ds(0, L)], pos_v)

  issue_idx(0, 0)
  wait_idx(0)
  issue_gathers(0)
  issue_idx(1, 1)

  vdesc.wait()
  pltpu.async_copy(vecbuf_v, out_hbm.at[pl.ds(base, ROWS_PER_W), 0], vsem)

  @pl.loop(0, NCHUNK // 2)
  def _pair(gg):
    for b in range(2):
      g = gg * 2 + b
      nb = 1 - b

      @pl.when(g + 1 < NCHUNK)
      def _():
        wait_idx(nb)

        @pl.when(g >= 1)
        def _():
          wait_out(nb)

        issue_gathers(nb)

      wait_gathers(b)

      @pl.when(g + 2 < NCHUNK)
      def _():
        issue_idx(g + 2, b)

      @pl.loop(0, L)
      def _pos(j):
        for q in range(HID // 16):
          p = pos_v[j, pl.ds(q * 16, 16)]
          for c in range(C):
            rows[b][c * L + j, pl.ds(q * 16, 16)] += p

      issue_out(g, b)

  wait_out(0)
  wait_out(1)
  pltpu.make_async_copy(vecbuf_v, out_hbm.at[pl.ds(base, ROWS_PER_W), 0],
                        vsem).wait()


def kernel(input_ids, vectors, word_table, pos_table):
  ids_flat = input_ids.reshape(B * L)
  pos_block = lax.slice_in_dim(pos_table, 1, MAXPOS + 1, axis=0)
  mesh = plsc.VectorSubcoreMesh(core_axis_name="c", subcore_axis_name="s",
                                num_cores=NC, num_subcores=NS)
  out = pl.kernel(
      _body,
      out_type=jax.ShapeDtypeStruct((B, MAXPOS + 1, HID), jnp.float32),
      mesh=mesh,
      compiler_params=pltpu.CompilerParams(use_tc_tiling_on_sc=False),
      scratch_types=[
          pltpu.VMEM((C * L,), jnp.int32),
          pltpu.VMEM((C * L,), jnp.int32),
          pltpu.VMEM((C * L, HID), jnp.float32),
          pltpu.VMEM((C * L, HID), jnp.float32),
          pltpu.VMEM((L, HID), jnp.float32),
          pltpu.VMEM((ROWS_PER_W, HID), jnp.float32),
          pltpu.SemaphoreType.DMA,
          pltpu.SemaphoreType.DMA,
          pltpu.SemaphoreType.DMA,
          pltpu.SemaphoreType.DMA,
          pltpu.SemaphoreType.DMA,
          pltpu.SemaphoreType.DMA,
          pltpu.SemaphoreType.DMA,
      ],
  )(ids_flat, vectors, word_table, pos_block)
  return out
